# baseline (speedup 1.0000x reference)
import jax
import jax.numpy as jnp
from jax import lax
from jax.experimental import pallas as pl
from jax.experimental.pallas import tpu as pltpu
from jax.experimental.pallas import tpu_sc as plsc

B = 4
S = 8192
D = 128
N = B * S

NC = 2
NS = 16
NW = NC * NS

PPW = S // NW
CH = 128
SUBS = PPW // CH
NCH = B * SUBS
NBUF = 4
LA = 2
LANES = 16
VPR = D // LANES


def _body(x_hbm, tok_hbm, pos_hbm, out_hbm, idx_v, pos_v, bufs, gsem, wsem):
    c = lax.axis_index("c")
    s = lax.axis_index("s")
    wid = s * NC + c
    p0 = wid * PPW

    for bt in range(B):
        for sub in range(SUBS):
            pltpu.sync_copy(
                x_hbm.at[bt, pl.ds(p0 + sub * CH, CH)],
                idx_v.at[bt * SUBS + sub],
            )

    def start_gather(j, jm):
        pltpu.async_copy(tok_hbm.at[idx_v.at[j]], bufs.at[jm], gsem)

    def wait_gather():
        pltpu.make_async_copy(tok_hbm.at[idx_v.at[0]], bufs.at[0], gsem).wait()

    def wait_writeback():
        pltpu.make_async_copy(bufs.at[0], out_hbm.at[0, pl.ds(0, CH)], wsem).wait()

    for j in range(LA):
        start_gather(j, j)

    pltpu.sync_copy(pos_hbm.at[pl.ds(p0, PPW)], pos_v)

    def chunk_body(j, carry):
        jm = lax.rem(j, NBUF)
        bt = lax.div(j, SUBS)
        sub = lax.rem(j, SUBS)
        jl = j + LA

        @pl.when(jl < NCH)
        def _():
            @pl.when(jl >= NBUF)
            def _():
                wait_writeback()

            start_gather(jl, lax.rem(jl, NBUF))

        wait_gather()

        buf = bufs.at[jm]
        prow = sub * CH

        def add_row(r, c2):
            for k in range(VPR):
                sl = pl.ds(k * LANES, LANES)
                buf[r, sl] += pos_v[prow + r, sl]
            return c2

        lax.fori_loop(0, CH, add_row, 0)
        pltpu.async_copy(buf, out_hbm.at[bt, pl.ds(p0 + sub * CH, CH)], wsem)
        return carry

    lax.fori_loop(0, NCH, chunk_body, 0)

    for _ in range(NBUF):
        wait_writeback()


@jax.jit
def _embed(x, tok_table, pos_table):
    run = pl.kernel(
        _body,
        out_type=jax.ShapeDtypeStruct((B, S, D), jnp.float32),
        mesh=plsc.VectorSubcoreMesh(
            core_axis_name="c", subcore_axis_name="s",
            num_cores=NC, num_subcores=NS,
        ),
        scratch_types=[
            pltpu.VMEM((NCH, CH), jnp.int32),
            pltpu.VMEM((PPW, D), jnp.float32),
            pltpu.VMEM((NBUF, CH, D), jnp.float32),
            pltpu.SemaphoreType.DMA,
            pltpu.SemaphoreType.DMA,
        ],
    )
    return run(x, tok_table, pos_table)


def kernel(x, tok_table, pos_table):
    return _embed(x.astype(jnp.int32), tok_table, pos_table)


# speedup vs baseline: 1.9259x; 1.9259x over previous
import jax
import jax.numpy as jnp
from jax import lax
from jax.experimental import pallas as pl
from jax.experimental.pallas import tpu as pltpu
from jax.experimental.pallas import tpu_sc as plsc

B = 4
S = 8192
D = 128
N = B * S

NC = 2
NS = 16
NW = NC * NS

PPW = S // NW
CH = 128
SUBS = PPW // CH
NCH = B * SUBS
NBUF = 4
LA = 2
LANES = 16
VPR = D // LANES


def _body(x_hbm, tok_hbm, pos_hbm, out_hbm, idx_v, pos_v, bufs, gsem, wsem):
    c = lax.axis_index("c")
    s = lax.axis_index("s")
    wid = s * NC + c
    p0 = wid * PPW

    for bt in range(B):
        for sub in range(SUBS):
            pltpu.sync_copy(
                x_hbm.at[bt, pl.ds(p0 + sub * CH, CH)],
                idx_v.at[bt * SUBS + sub],
            )

    def start_gather(j):
        return pltpu.async_copy(tok_hbm.at[idx_v.at[j]], bufs.at[j % NBUF], gsem)

    gathers = {j: start_gather(j) for j in range(LA)}
    writebacks = {}

    pltpu.sync_copy(pos_hbm.at[pl.ds(p0, PPW)], pos_v)

    for j in range(NCH):
        bt, sub = j // SUBS, j % SUBS
        if j + LA < NCH:
            if j + LA >= NBUF:
                writebacks.pop(j + LA - NBUF).wait()
            gathers[j + LA] = start_gather(j + LA)
        gathers.pop(j).wait()

        buf = bufs.at[j % NBUF]
        prow = sub * CH

        def add_row(r, c2):
            for k in range(VPR):
                sl = pl.ds(k * LANES, LANES)
                buf[r, sl] += pos_v[prow + r, sl]
            return c2

        lax.fori_loop(0, CH, add_row, 0)
        writebacks[j] = pltpu.async_copy(
            buf, out_hbm.at[bt, pl.ds(p0 + sub * CH, CH)], wsem)

    for j in sorted(writebacks):
        writebacks.pop(j).wait()


@jax.jit
def _embed(x, tok_table, pos_table):
    run = pl.kernel(
        _body,
        out_type=jax.ShapeDtypeStruct((B, S, D), jnp.float32),
        mesh=plsc.VectorSubcoreMesh(
            core_axis_name="c", subcore_axis_name="s",
            num_cores=NC, num_subcores=NS,
        ),
        scratch_types=[
            pltpu.VMEM((NCH, CH), jnp.int32),
            pltpu.VMEM((PPW, D), jnp.float32),
            pltpu.VMEM((NBUF, CH, D), jnp.float32),
            pltpu.SemaphoreType.DMA,
            pltpu.SemaphoreType.DMA,
        ],
    )
    return run(x, tok_table, pos_table)


def kernel(x, tok_table, pos_table):
    return _embed(x.astype(jnp.int32), tok_table, pos_table)


# speedup vs baseline: 2.0425x; 1.0605x over previous
import jax
import jax.numpy as jnp
from jax import lax
from jax.experimental import pallas as pl
from jax.experimental.pallas import tpu as pltpu
from jax.experimental.pallas import tpu_sc as plsc

B = 4
S = 8192
D = 128
N = B * S

NC = 2
NS = 16
NW = NC * NS

PPW = S // NW
CH = 128
SUBS = PPW // CH
NCH = B * SUBS
NBUF = 5
LA = 3
LANES = 16
VPR = D // LANES


def _body(x_hbm, tok_hbm, pos_hbm, out_hbm, idx_v, pos_v, bufs, gsem, wsem):
    c = lax.axis_index("c")
    s = lax.axis_index("s")
    wid = s * NC + c
    p0 = wid * PPW

    for bt in range(B):
        pltpu.sync_copy(
            x_hbm.at[pl.ds(bt * (S // CH) + wid * SUBS, SUBS)],
            idx_v.at[pl.ds(bt * SUBS, SUBS)],
        )

    def out_row(j):
        bt, sub = j // SUBS, j % SUBS
        return bt * S + p0 + sub * CH

    def start_gather(j):
        return pltpu.async_copy(tok_hbm.at[idx_v.at[j]], bufs.at[j % NBUF], gsem)

    gathers = {j: start_gather(j) for j in range(LA)}
    writebacks = {}

    pltpu.sync_copy(pos_hbm.at[pl.ds(p0, PPW)], pos_v)

    for j in range(NCH):
        if j + LA < NCH:
            if j + LA >= NBUF:
                writebacks.pop(j + LA - NBUF).wait()
            gathers[j + LA] = start_gather(j + LA)
        gathers.pop(j).wait()

        buf = bufs.at[j % NBUF]
        prow = (j % SUBS) * CH

        def add_row(r, carry):
            for k in range(VPR):
                sl = pl.ds(k * LANES, LANES)
                buf[r, sl] += pos_v[prow + r, sl]
            return carry

        lax.fori_loop(0, CH, add_row, 0)
        writebacks[j] = pltpu.async_copy(buf, out_hbm.at[pl.ds(out_row(j), CH)], wsem)

    for j in sorted(writebacks):
        writebacks.pop(j).wait()


@jax.jit
def _embed(x_flat, tok_table, pos_table):
    run = pl.kernel(
        _body,
        out_type=jax.ShapeDtypeStruct((N, D), jnp.float32),
        mesh=plsc.VectorSubcoreMesh(
            core_axis_name="c", subcore_axis_name="s",
            num_cores=NC, num_subcores=NS,
        ),
        scratch_types=[
            pltpu.VMEM((NCH, CH), jnp.int32),
            pltpu.VMEM((PPW, D), jnp.float32),
            pltpu.VMEM((NBUF, CH, D), jnp.float32),
            pltpu.SemaphoreType.DMA,
            pltpu.SemaphoreType.DMA,
        ],
    )
    return run(x_flat.reshape(N // CH, CH), tok_table, pos_table)


def kernel(x, tok_table, pos_table):
    x_flat = x.reshape(-1).astype(jnp.int32)
    out = _embed(x_flat, tok_table, pos_table)
    return out.reshape(B, S, D)
